# baseline (speedup 1.0000x reference)
import functools

import jax
import jax.numpy as jnp
import numpy as np
from jax import lax
from jax.experimental import pallas as pl
from jax.experimental.pallas import tpu as pltpu
from jax.experimental.pallas import tpu_sc as plsc

VOCAB = 100000
D_MODEL = 768
MAX_LEN = 8192
BATCH = 4
SEQ = 2048

NUM_CORES = 2
NUM_SUBCORES = 16
NUM_WORKERS = NUM_CORES * NUM_SUBCORES
S_PER_W = SEQ // NUM_WORKERS
LANES = 16
GROUPS = D_MODEL // LANES


def _pos_encoding_np(max_len, d_model):
    pos = np.arange(max_len, dtype=np.float32)[:, None]
    i = np.arange(d_model, dtype=np.float32)[None, :]
    angle_rates = 1.0 / np.power(10000.0, (2.0 * np.floor(i / 2.0)) / d_model)
    angles = pos * angle_rates
    pe = np.zeros((max_len, d_model), dtype=np.float32)
    pe[:, 0::2] = np.sin(angles[:, 0::2])
    pe[:, 1::2] = np.cos(angles[:, 1::2])
    return pe


_PE = _pos_encoding_np(SEQ, D_MODEL)


def _emb_kernel(x_hbm, table_hbm, pe_hbm, out_hbm, idx_v, rows_v, pe_v, sem):
    wid = lax.axis_index("s") * NUM_CORES + lax.axis_index("c")
    s0 = wid * S_PER_W

    pltpu.sync_copy(pe_hbm.at[pl.ds(s0, S_PER_W), :], pe_v)

    for b in range(BATCH):
        pltpu.sync_copy(x_hbm.at[b, pl.ds(s0, S_PER_W)], idx_v)
        pltpu.async_copy(table_hbm.at[idx_v], rows_v, sem).wait()

        def _add_row(t, _):
            for g in range(GROUPS):
                sl = pl.ds(g * LANES, LANES)
                rows_v[t, sl] = rows_v[t, sl] + pe_v[t, sl]
            return _

        lax.fori_loop(0, S_PER_W, _add_row, 0)

        pltpu.sync_copy(rows_v, out_hbm.at[b, pl.ds(s0, S_PER_W), :])


@jax.jit
def kernel(x, tok_table):
    mesh = plsc.VectorSubcoreMesh(core_axis_name="c", subcore_axis_name="s")
    call = pl.kernel(
        _emb_kernel,
        out_type=jax.ShapeDtypeStruct((BATCH, SEQ, D_MODEL), jnp.float32),
        mesh=mesh,
        scratch_types=[
            pltpu.VMEM((S_PER_W,), jnp.int32),
            pltpu.VMEM((S_PER_W, D_MODEL), jnp.float32),
            pltpu.VMEM((S_PER_W, D_MODEL), jnp.float32),
            pltpu.SemaphoreType.DMA,
        ],
    )
    return call(x, tok_table, jnp.asarray(_PE))
